# speedup vs baseline: 6.4443x; 6.4443x over previous
import jax
import jax.numpy as jnp
from jax import lax
from jax.experimental import pallas as pl
from jax.experimental.pallas import tpu as pltpu
from jax.experimental.pallas import tpu_sc as plsc

N = 320000
D = 128
S = 2048

NUM_CORES = 2
NUM_SUBCORES = 16
NUM_WORKERS = NUM_CORES * NUM_SUBCORES

CHUNK_ROWS = 512
IDX_ROWS = CHUNK_ROWS // 128
NUM_CHUNKS = N // CHUNK_ROWS
BASE_STEPS = NUM_CHUNKS // NUM_WORKERS
REMAINDER = NUM_CHUNKS - BASE_STEPS * NUM_WORKERS


def _sc_partials_body(nf_hbm, batch_hbm, part_hbm, rows_v, idx_v, acc):
    c = lax.axis_index("c")
    s = lax.axis_index("s")
    w = s * NUM_CORES + c

    def zero_row(i, _):
        for j in range(D // 16):
            rows_v[i, pl.ds(j * 16, 16)] = jnp.zeros((16,), jnp.float32)
        return 0

    lax.fori_loop(0, 128, zero_row, 0)
    pltpu.sync_copy(rows_v.at[pl.ds(0, 128)], acc.at[pl.ds(s * 128, 128)])
    plsc.subcore_barrier()

    def process(chunk):
        pltpu.sync_copy(nf_hbm.at[pl.ds(chunk * CHUNK_ROWS, CHUNK_ROWS)], rows_v)
        pltpu.sync_copy(batch_hbm.at[pl.ds(chunk * IDX_ROWS, IDX_ROWS)], idx_v)
        for j in range(IDX_ROWS):
            pltpu.sync_copy(
                rows_v.at[pl.ds(j * 128, 128)], acc.at[idx_v.at[j]], add=True
            )

    def step(t, _):
        process(w + NUM_WORKERS * t)
        return 0

    lax.fori_loop(0, BASE_STEPS, step, 0)

    @pl.when(w < REMAINDER)
    def _():
        process(BASE_STEPS * NUM_WORKERS + w)

    plsc.subcore_barrier()
    pltpu.sync_copy(acc.at[pl.ds(s * 128, 128)], rows_v.at[pl.ds(0, 128)])
    pltpu.sync_copy(rows_v.at[pl.ds(0, 128)], part_hbm.at[c, pl.ds(s * 128, 128)])


_sc_partials = pl.kernel(
    _sc_partials_body,
    out_type=jax.ShapeDtypeStruct((NUM_CORES, S, D), jnp.float32),
    mesh=plsc.VectorSubcoreMesh(core_axis_name="c", subcore_axis_name="s"),
    scratch_types=[
        pltpu.VMEM((CHUNK_ROWS, D), jnp.float32),
        pltpu.VMEM((IDX_ROWS, 128), jnp.int32),
        pltpu.VMEM_SHARED((S, D), jnp.float32),
    ],
)


def _combine_body(p_ref, o_ref):
    o_ref[...] = p_ref[0] + p_ref[1]


_combine = pl.pallas_call(
    _combine_body,
    out_shape=jax.ShapeDtypeStruct((S, D), jnp.float32),
)


def kernel(node_features, batch, ptr):
    del ptr
    batch2d = batch.reshape(N // 128, 128)
    partials = _sc_partials(node_features, batch2d)
    return _combine(partials)


# speedup vs baseline: 8.5354x; 1.3245x over previous
import jax
import jax.numpy as jnp
from jax import lax
from jax.experimental import pallas as pl
from jax.experimental.pallas import tpu as pltpu
from jax.experimental.pallas import tpu_sc as plsc

N = 320000
D = 128
S = 2048

NUM_CORES = 2
NUM_SUBCORES = 16
NUM_WORKERS = NUM_CORES * NUM_SUBCORES

CHUNK_ROWS = 256
IDX_ROWS = CHUNK_ROWS // 128
NUM_CHUNKS = N // CHUNK_ROWS
BASE_STEPS = NUM_CHUNKS // NUM_WORKERS
REMAINDER = NUM_CHUNKS - BASE_STEPS * NUM_WORKERS
MAX_STEPS = BASE_STEPS + (1 if REMAINDER else 0)


def _sc_partials_body(nf_hbm, batch_hbm, part_hbm, rows_v, idx_v, acc, ld_sem):
    c = lax.axis_index("c")
    s = lax.axis_index("s")
    w = s * NUM_CORES + c

    def zero_row(i, _):
        for j in range(D // 16):
            rows_v[0, i, pl.ds(j * 16, 16)] = jnp.zeros((16,), jnp.float32)
        return 0

    lax.fori_loop(0, 128, zero_row, 0)
    pltpu.sync_copy(rows_v.at[0, pl.ds(0, 128)], acc.at[pl.ds(s * 128, 128)])
    plsc.subcore_barrier()

    n = jnp.where(w < REMAINDER, BASE_STEPS + 1, BASE_STEPS)

    def chunk_of(t):
        return jnp.where(
            t < BASE_STEPS, w + NUM_WORKERS * t, BASE_STEPS * NUM_WORKERS + w
        )

    def issue_loads(t, p):
        chunk = chunk_of(t)
        pltpu.async_copy(
            nf_hbm.at[pl.ds(chunk * CHUNK_ROWS, CHUNK_ROWS)],
            rows_v.at[p],
            ld_sem.at[p],
        )
        pltpu.async_copy(
            batch_hbm.at[pl.ds(chunk * IDX_ROWS, IDX_ROWS)],
            idx_v.at[p],
            ld_sem.at[p],
        )

    def wait_loads(p):
        pltpu.make_async_copy(
            nf_hbm.at[pl.ds(0, CHUNK_ROWS)], rows_v.at[p], ld_sem.at[p]
        ).wait()
        pltpu.make_async_copy(
            batch_hbm.at[pl.ds(0, IDX_ROWS)], idx_v.at[p], ld_sem.at[p]
        ).wait()

    def scatter(p):
        for j in range(IDX_ROWS):
            pltpu.sync_copy(
                rows_v.at[p, pl.ds(j * 128, 128)], acc.at[idx_v.at[p, j]], add=True
            )

    issue_loads(0, 0)

    def two_steps(i, _):
        t0 = 2 * i
        t1 = 2 * i + 1

        @pl.when(t0 < n)
        def _():
            wait_loads(0)

            @pl.when(t1 < n)
            def _():
                issue_loads(t1, 1)

            scatter(0)

        @pl.when(t1 < n)
        def _():
            wait_loads(1)

            @pl.when(t1 + 1 < n)
            def _():
                issue_loads(t1 + 1, 0)

            scatter(1)

        return 0

    lax.fori_loop(0, (MAX_STEPS + 1) // 2, two_steps, 0)

    plsc.subcore_barrier()
    pltpu.sync_copy(acc.at[pl.ds(s * 128, 128)], rows_v.at[0, pl.ds(0, 128)])
    pltpu.sync_copy(
        rows_v.at[0, pl.ds(0, 128)], part_hbm.at[c, pl.ds(s * 128, 128)]
    )


_sc_partials = pl.kernel(
    _sc_partials_body,
    out_type=jax.ShapeDtypeStruct((NUM_CORES, S, D), jnp.float32),
    mesh=plsc.VectorSubcoreMesh(core_axis_name="c", subcore_axis_name="s"),
    scratch_types=[
        pltpu.VMEM((2, CHUNK_ROWS, D), jnp.float32),
        pltpu.VMEM((2, IDX_ROWS, 128), jnp.int32),
        pltpu.VMEM_SHARED((S, D), jnp.float32),
        pltpu.SemaphoreType.DMA((2,)),
    ],
)


def _combine_body(p_ref, o_ref):
    o_ref[...] = p_ref[0] + p_ref[1]


_combine = pl.pallas_call(
    _combine_body,
    out_shape=jax.ShapeDtypeStruct((S, D), jnp.float32),
)


def kernel(node_features, batch, ptr):
    del ptr
    batch2d = batch.reshape(N // 128, 128)
    partials = _sc_partials(node_features, batch2d)
    return _combine(partials)
